# baseline (speedup 1.0000x reference)
import numpy as np
import jax
import jax.numpy as jnp
from jax import lax
from jax.experimental import pallas as pl
from jax.experimental.pallas import tpu as pltpu
from jax.experimental.pallas import tpu_sc as plsc

N_SYMBOLS = 100000
EMBED = 1024
BATCH = 4
SEQ = 2048

NC = 2
NS = 16
NW = NC * NS
SEQ_PER_W = SEQ // NW
CH = 8
NCH = SEQ_PER_W // CH
STEPS = NCH * BATCH
NBUF = 10
NPE = 2
LANES = 16


def _position_encoding() -> np.ndarray:
    pos = np.arange(SEQ, dtype=np.float32)[:, None]
    i = np.arange(EMBED // 2, dtype=np.float32)[None, :]
    rates = np.power(np.float32(10000.0), -(2.0 * i) / np.float32(EMBED))
    ang = pos * rates
    return np.concatenate([np.sin(ang), np.cos(ang)], axis=-1).astype(np.float32)


_PE = _position_encoding()


def _sc_body(table_hbm, idx_hbm, pe_hbm, out_hbm,
             idx_v, pe_v, rows_v, gsem, osem, pesem):
    c = lax.axis_index("c")
    s = lax.axis_index("s")
    wid = s * NC + c
    seq_base = wid * SEQ_PER_W

    for b in range(BATCH):
        pltpu.sync_copy(idx_hbm.at[b, pl.ds(seq_base, SEQ_PER_W)], idx_v.at[b])

    def fire_pe_load(co):
        return pltpu.async_copy(
            pe_hbm.at[pl.ds(seq_base + co * CH, CH)],
            pe_v.at[co % NPE], pesem.at[co % NPE])

    def fire_gather(k):
        co, b = divmod(k, BATCH)
        p = k % NBUF
        return pltpu.async_copy(
            table_hbm.at[idx_v.at[b, pl.ds(co * CH, CH)]],
            rows_v.at[p], gsem.at[p])

    pecp = {0: fire_pe_load(0), 1: fire_pe_load(1)}
    gcp = {k: fire_gather(k) for k in range(NBUF)}

    scp = {}
    for k in range(STEPS):
        p = k % NBUF
        co, b = divmod(k, BATCH)

        if k % BATCH == 0:
            pecp[co].wait()
            if 1 <= co <= NCH - NPE:
                pecp[co + 1] = fire_pe_load(co + 1)

        gcp[k].wait()

        UNR = 16

        def _row(r, carry):
            def _blk(jb, carry2):
                base = jb * (UNR * LANES)
                for u in range(UNR):
                    sl = pl.ds(base + u * LANES, LANES)
                    plsc.addupdate(rows_v.at[p, r, sl], pe_v[co % NPE, r, sl])
                return carry2
            return lax.fori_loop(0, EMBED // (UNR * LANES), _blk, carry)

        lax.fori_loop(0, CH, _row, 0)

        scp[k] = pltpu.async_copy(
            rows_v.at[p], out_hbm.at[b, pl.ds(seq_base + co * CH, CH)],
            osem.at[p])

        if k >= 2:
            scp[k - 2].wait()
            if k + NBUF - 2 < STEPS:
                gcp[k + NBUF - 2] = fire_gather(k + NBUF - 2)

    for k in range(STEPS - 2, STEPS):
        scp[k].wait()


@jax.jit
def _embed(table, idx, pe):
    mesh = plsc.VectorSubcoreMesh(core_axis_name="c", subcore_axis_name="s")
    f = pl.kernel(
        _sc_body,
        mesh=mesh,
        out_type=jax.ShapeDtypeStruct((BATCH, SEQ, EMBED), jnp.float32),
        scratch_types=[
            pltpu.VMEM((BATCH, SEQ_PER_W), jnp.int32),
            pltpu.VMEM((NPE, CH, EMBED), jnp.float32),
            pltpu.VMEM((NBUF, CH, EMBED), jnp.float32),
            pltpu.SemaphoreType.DMA((NBUF,)),
            pltpu.SemaphoreType.DMA((NBUF,)),
            pltpu.SemaphoreType.DMA((NPE,)),
        ],
    )
    return f(table, idx, pe)


def kernel(inputs, embedding_table):
    idx = inputs.astype(jnp.int32)
    pe = jnp.asarray(_PE)
    return _embed(embedding_table, idx, pe)


# speedup vs baseline: 1.1100x; 1.1100x over previous
import numpy as np
import jax
import jax.numpy as jnp
from jax import lax
from jax.experimental import pallas as pl
from jax.experimental.pallas import tpu as pltpu
from jax.experimental.pallas import tpu_sc as plsc

N_SYMBOLS = 100000
EMBED = 1024
BATCH = 4
SEQ = 2048

NC = 2
NS = 16
NW = NC * NS
SEQ_PER_W = SEQ // NW
CH = 16
NCH = SEQ_PER_W // CH
STEPS = NCH * BATCH
NBUF = 5
NPE = 2
LANES = 16


def _position_encoding() -> np.ndarray:
    pos = np.arange(SEQ, dtype=np.float32)[:, None]
    i = np.arange(EMBED // 2, dtype=np.float32)[None, :]
    rates = np.power(np.float32(10000.0), -(2.0 * i) / np.float32(EMBED))
    ang = pos * rates
    return np.concatenate([np.sin(ang), np.cos(ang)], axis=-1).astype(np.float32)


_PE = _position_encoding()


def _sc_body(table_hbm, idx_hbm, pe_hbm, out_hbm,
             idx_v, pe_v, rows_v, gsem, osem, pesem):
    c = lax.axis_index("c")
    s = lax.axis_index("s")
    wid = s * NC + c
    seq_base = wid * SEQ_PER_W

    for b in range(BATCH):
        pltpu.sync_copy(idx_hbm.at[b, pl.ds(seq_base, SEQ_PER_W)], idx_v.at[b])

    def fire_pe_load(co):
        return pltpu.async_copy(
            pe_hbm.at[pl.ds(seq_base + co * CH, CH)],
            pe_v.at[co % NPE], pesem.at[co % NPE])

    def fire_gather(k):
        co, b = divmod(k, BATCH)
        p = k % NBUF
        return pltpu.async_copy(
            table_hbm.at[idx_v.at[b, pl.ds(co * CH, CH)]],
            rows_v.at[p], gsem.at[p])

    pecp = {0: fire_pe_load(0), 1: fire_pe_load(1)}
    gcp = {k: fire_gather(k) for k in range(NBUF)}

    scp = {}
    for k in range(STEPS):
        p = k % NBUF
        co, b = divmod(k, BATCH)

        if k % BATCH == 0:
            pecp[co].wait()
            if 1 <= co <= NCH - NPE:
                pecp[co + 1] = fire_pe_load(co + 1)

        gcp[k].wait()

        def _row(r, carry):
            for j in range(EMBED // LANES):
                sl = pl.ds(j * LANES, LANES)
                plsc.addupdate(rows_v.at[p, r, sl], pe_v[co % NPE, r, sl])
            return carry

        lax.fori_loop(0, CH, _row, 0)

        scp[k] = pltpu.async_copy(
            rows_v.at[p], out_hbm.at[b, pl.ds(seq_base + co * CH, CH)],
            osem.at[p])

        if k >= 1:
            scp[k - 1].wait()
            if k + 4 < STEPS:
                gcp[k + 4] = fire_gather(k + 4)

    scp[STEPS - 1].wait()


@jax.jit
def _embed(table, idx, pe):
    mesh = plsc.VectorSubcoreMesh(core_axis_name="c", subcore_axis_name="s")
    f = pl.kernel(
        _sc_body,
        mesh=mesh,
        out_type=jax.ShapeDtypeStruct((BATCH, SEQ, EMBED), jnp.float32),
        scratch_types=[
            pltpu.VMEM((BATCH, SEQ_PER_W), jnp.int32),
            pltpu.VMEM((NPE, CH, EMBED), jnp.float32),
            pltpu.VMEM((NBUF, CH, EMBED), jnp.float32),
            pltpu.SemaphoreType.DMA((NBUF,)),
            pltpu.SemaphoreType.DMA((NBUF,)),
            pltpu.SemaphoreType.DMA((NPE,)),
        ],
    )
    return f(table, idx, pe)


def kernel(inputs, embedding_table):
    idx = inputs.astype(jnp.int32)
    pe = jnp.asarray(_PE)
    return _embed(embedding_table, idx, pe)
